# baseline (speedup 1.0000x reference)
import functools

import jax
import jax.numpy as jnp
from jax import lax
from jax.experimental import pallas as pl
from jax.experimental.pallas import tpu as pltpu
from jax.experimental.pallas import tpu_sc as plsc

NUM_USERS = 100000
NUM_ITEMS = 100000
EMB = 64
T = 3
BATCH = 16384
LAMB = 0.5

NC = 2
NS = 16
NW = NC * NS
BPW = BATCH // NW
CHUNK = 128
NCHUNK = BPW // CHUNK
PW = 2 * EMB

_SC_MESH = plsc.VectorSubcoreMesh(core_axis_name="c", subcore_axis_name="s")


@functools.partial(
    pl.kernel,
    out_type=jax.ShapeDtypeStruct((BATCH, PW), jnp.float32),
    mesh=_SC_MESH,
    scratch_types=[
        pltpu.VMEM((NCHUNK, CHUNK), jnp.int32),
        pltpu.VMEM((BPW, PW), jnp.float32),
        pltpu.SemaphoreType.DMA,
    ],
    compiler_params=pltpu.CompilerParams(use_tc_tiling_on_sc=False),
)
def _sc_gather(plane, idx, out, idx_v, rows_v, sem):
    wid = lax.axis_index("s") * NC + lax.axis_index("c")
    pltpu.sync_copy(idx.at[wid], idx_v)
    copies = [
        pltpu.async_copy(
            plane.at[idx_v.at[j]],
            rows_v.at[pl.ds(j * CHUNK, CHUNK)],
            sem,
        )
        for j in range(NCHUNK)
    ]
    for c in copies:
        c.wait()
    pltpu.sync_copy(rows_v, out.at[pl.ds(wid * BPW, BPW)])


TN = 4096


def _transpose_body(at_ref, bt_ref, out_ref):
    def rne(v):
        u = lax.bitcast_convert_type(v, jnp.uint32)
        r = u + jnp.uint32(0x7FFF) + ((u >> 16) & jnp.uint32(1))
        return r & jnp.uint32(0xFFFF0000)

    def pk(a, b):
        w = rne(a) | (rne(b) >> 16)
        return jnp.transpose(lax.bitcast_convert_type(w, jnp.float32),
                             (1, 0))

    out_ref[...] = jnp.concatenate(
        [pk(at_ref[...], bt_ref[0]), pk(bt_ref[1], bt_ref[2])], axis=1)


def _transpose_side(at, bt):
    nb = (NUM_USERS + TN - 1) // TN
    return pl.pallas_call(
        _transpose_body,
        grid=(nb,),
        in_specs=[
            pl.BlockSpec((EMB, TN), lambda i: (0, i)),
            pl.BlockSpec((T, EMB, TN), lambda i: (0, 0, i)),
        ],
        out_specs=pl.BlockSpec((TN, PW), lambda i: (i, 0)),
        out_shape=jax.ShapeDtypeStruct((NUM_USERS, PW), jnp.float32),
    )(at, bt)


BLK = 2048


def _tc_body(ru_ref, ri_ref, m_ref, out_ref):
    def unpk(r, k):
        w = lax.bitcast_convert_type(
            r[:, k * EMB:(k + 1) * EMB], jnp.uint32)
        hi = lax.bitcast_convert_type(w & jnp.uint32(0xFFFF0000),
                                      jnp.float32)
        lo = lax.bitcast_convert_type(w << 16, jnp.float32)
        return hi, lo

    ru = ru_ref[...]
    ri = ri_ref[...]
    u, p0 = unpk(ru, 0)
    p1, p2 = unpk(ru, 1)
    i, s0 = unpk(ri, 0)
    s1, s2 = unpk(ri, 1)
    p = (p0, p1, p2)
    s = (s0, s1, s2)
    acc = LAMB * jnp.sum(u * i, axis=-1, keepdims=True)
    w = (1.0 - LAMB) / T
    for t in range(T):
        proj = lax.dot_general(
            p[t].astype(jnp.bfloat16),
            m_ref[t].astype(jnp.bfloat16),
            (((1,), (0,)), ((), ())),
            preferred_element_type=jnp.float32,
        )
        acc = acc + w * jnp.sum(proj * s[t], axis=-1, keepdims=True)
    out_ref[...] = acc


def kernel(user_idx, item_idx, user_latent, item_latent, s_item_list,
           user_mean_emb, M_t):
    ui = user_idx.astype(jnp.int32).reshape(NW, NCHUNK, CHUNK)
    ii = item_idx.astype(jnp.int32).reshape(NW, NCHUNK, CHUNK)
    ult = jnp.swapaxes(user_latent, 0, 1)
    ilt = jnp.swapaxes(item_latent, 0, 1)
    umt = jnp.transpose(user_mean_emb, (0, 2, 1))
    sst = jnp.transpose(s_item_list, (0, 2, 1))

    plane_u = _transpose_side(ult, umt)
    rows_u = _sc_gather(plane_u, ui)
    plane_i = _transpose_side(ilt, sst)
    rows_i = _sc_gather(plane_i, ii)

    score2 = pl.pallas_call(
        _tc_body,
        grid=(BATCH // BLK,),
        in_specs=[
            pl.BlockSpec((BLK, PW), lambda i: (i, 0)),
            pl.BlockSpec((BLK, PW), lambda i: (i, 0)),
            pl.BlockSpec((T, EMB, EMB), lambda i: (0, 0, 0)),
        ],
        out_specs=pl.BlockSpec((BLK, 1), lambda i: (i, 0)),
        out_shape=jax.ShapeDtypeStruct((BATCH, 1), jnp.float32),
    )(rows_u, rows_i, M_t)
    return score2[:, 0]


# speedup vs baseline: 1.0659x; 1.0659x over previous
import functools

import jax
import jax.numpy as jnp
from jax import lax
from jax.experimental import pallas as pl
from jax.experimental.pallas import tpu as pltpu
from jax.experimental.pallas import tpu_sc as plsc

NUM_USERS = 100000
NUM_ITEMS = 100000
EMB = 64
T = 3
BATCH = 16384
LAMB = 0.5

NC = 2
NS = 16
NW = NC * NS
BPW = BATCH // NW
CHUNK = 128
NCHUNK = BPW // CHUNK
PW = 2 * EMB

_SC_MESH = plsc.VectorSubcoreMesh(core_axis_name="c", subcore_axis_name="s")


@functools.partial(
    pl.kernel,
    out_type=jax.ShapeDtypeStruct((2, BATCH, PW), jnp.float32),
    mesh=_SC_MESH,
    scratch_types=[
        pltpu.VMEM((NCHUNK, CHUNK), jnp.int32),
        pltpu.VMEM((BPW, PW), jnp.float32),
        pltpu.SemaphoreType.DMA,
    ],
    compiler_params=pltpu.CompilerParams(use_tc_tiling_on_sc=False),
)
def _sc_gather(planes, idx, out, idx_v, rows_v, sem):
    wid = lax.axis_index("s") * NC + lax.axis_index("c")
    for g in range(2):
        pltpu.sync_copy(idx.at[g, wid], idx_v)
        copies = [
            pltpu.async_copy(
                planes.at[idx_v.at[j]],
                rows_v.at[pl.ds(j * CHUNK, CHUNK)],
                sem,
            )
            for j in range(NCHUNK)
        ]
        for c in copies:
            c.wait()
        pltpu.sync_copy(rows_v, out.at[g, pl.ds(wid * BPW, BPW)])


TN = 4096


def _transpose_body(ult_ref, ilt_ref, umt_ref, sst_ref, out_ref):
    def rne(v):
        u = lax.bitcast_convert_type(v, jnp.uint32)
        r = u + jnp.uint32(0x7FFF) + ((u >> 16) & jnp.uint32(1))
        return r & jnp.uint32(0xFFFF0000)

    def pk(a, b):
        w = rne(a) | (rne(b) >> 16)
        return jnp.transpose(lax.bitcast_convert_type(w, jnp.float32),
                             (1, 0))

    out_ref[0] = jnp.concatenate(
        [pk(ult_ref[...], umt_ref[0]), pk(umt_ref[1], umt_ref[2])], axis=1)
    out_ref[1] = jnp.concatenate(
        [pk(ilt_ref[...], sst_ref[0]), pk(sst_ref[1], sst_ref[2])], axis=1)


def _transpose_all(ult, ilt, umt, sst):
    nb = (NUM_USERS + TN - 1) // TN
    return pl.pallas_call(
        _transpose_body,
        grid=(nb,),
        in_specs=[
            pl.BlockSpec((EMB, TN), lambda i: (0, i)),
            pl.BlockSpec((EMB, TN), lambda i: (0, i)),
            pl.BlockSpec((T, EMB, TN), lambda i: (0, 0, i)),
            pl.BlockSpec((T, EMB, TN), lambda i: (0, 0, i)),
        ],
        out_specs=pl.BlockSpec((2, TN, PW), lambda i: (0, i, 0)),
        out_shape=jax.ShapeDtypeStruct((2, NUM_USERS, PW), jnp.float32),
    )(ult, ilt, umt, sst)


BLK = 2048


def _tc_body(rows_ref, m_ref, out_ref):
    def unpk(r, k):
        w = lax.bitcast_convert_type(
            r[:, k * EMB:(k + 1) * EMB], jnp.uint32)
        hi = lax.bitcast_convert_type(w & jnp.uint32(0xFFFF0000),
                                      jnp.float32)
        lo = lax.bitcast_convert_type(w << 16, jnp.float32)
        return hi, lo

    ru = rows_ref[0]
    ri = rows_ref[1]
    u, p0 = unpk(ru, 0)
    p1, p2 = unpk(ru, 1)
    i, s0 = unpk(ri, 0)
    s1, s2 = unpk(ri, 1)
    p = (p0, p1, p2)
    s = (s0, s1, s2)
    acc = LAMB * jnp.sum(u * i, axis=-1, keepdims=True)
    w = (1.0 - LAMB) / T
    for t in range(T):
        proj = lax.dot_general(
            p[t].astype(jnp.bfloat16),
            m_ref[t].astype(jnp.bfloat16),
            (((1,), (0,)), ((), ())),
            preferred_element_type=jnp.float32,
        )
        acc = acc + w * jnp.sum(proj * s[t], axis=-1, keepdims=True)
    out_ref[...] = acc


def kernel(user_idx, item_idx, user_latent, item_latent, s_item_list,
           user_mean_emb, M_t):
    ui = user_idx.astype(jnp.int32)
    ii = item_idx.astype(jnp.int32)
    ult = jnp.swapaxes(user_latent, 0, 1)
    ilt = jnp.swapaxes(item_latent, 0, 1)
    umt = jnp.transpose(user_mean_emb, (0, 2, 1))
    sst = jnp.transpose(s_item_list, (0, 2, 1))
    planes = _transpose_all(ult, ilt, umt, sst)
    planes_flat = planes.reshape(2 * NUM_USERS, PW)
    idx_all = jnp.concatenate(
        [ui[None], ii[None] + jnp.int32(NUM_USERS)], axis=0
    ).reshape(2, NW, NCHUNK, CHUNK)

    rows = _sc_gather(planes_flat, idx_all)

    score2 = pl.pallas_call(
        _tc_body,
        grid=(BATCH // BLK,),
        in_specs=[
            pl.BlockSpec((2, BLK, PW), lambda i: (0, i, 0)),
            pl.BlockSpec((T, EMB, EMB), lambda i: (0, 0, 0)),
        ],
        out_specs=pl.BlockSpec((BLK, 1), lambda i: (i, 0)),
        out_shape=jax.ShapeDtypeStruct((BATCH, 1), jnp.float32),
    )(rows, M_t)
    return score2[:, 0]


# speedup vs baseline: 1.0741x; 1.0077x over previous
import functools

import jax
import jax.numpy as jnp
from jax import lax
from jax.experimental import pallas as pl
from jax.experimental.pallas import tpu as pltpu
from jax.experimental.pallas import tpu_sc as plsc

NUM_USERS = 100000
NUM_ITEMS = 100000
EMB = 64
T = 3
BATCH = 16384
LAMB = 0.5

NC = 2
NS = 16
NW = NC * NS
BPW = BATCH // NW
CHUNK = 128
NCHUNK = BPW // CHUNK
PW = 2 * EMB

_SC_MESH = plsc.VectorSubcoreMesh(core_axis_name="c", subcore_axis_name="s")


@functools.partial(
    pl.kernel,
    out_type=jax.ShapeDtypeStruct((2, BATCH, PW), jnp.float32),
    mesh=_SC_MESH,
    scratch_types=[
        pltpu.VMEM((NCHUNK, CHUNK), jnp.int32),
        pltpu.VMEM((BPW, PW), jnp.float32),
        pltpu.SemaphoreType.DMA,
    ],
    compiler_params=pltpu.CompilerParams(use_tc_tiling_on_sc=False),
)
def _sc_gather(planes, idx, out, idx_v, rows_v, sem):
    wid = lax.axis_index("s") * NC + lax.axis_index("c")
    for g in range(2):
        pltpu.sync_copy(idx.at[g, wid], idx_v)
        copies = [
            pltpu.async_copy(
                planes.at[idx_v.at[j]],
                rows_v.at[pl.ds(j * CHUNK, CHUNK)],
                sem,
            )
            for j in range(NCHUNK)
        ]
        for c in copies:
            c.wait()
        pltpu.sync_copy(rows_v, out.at[g, pl.ds(wid * BPW, BPW)])


TN = 4096


def _transpose_body(ult_ref, ilt_ref, umt_ref, sst_ref, out_ref):
    def rne(v):
        u = lax.bitcast_convert_type(v, jnp.uint32)
        r = u + jnp.uint32(0x7FFF) + ((u >> 16) & jnp.uint32(1))
        return r & jnp.uint32(0xFFFF0000)

    def pk(a, b):
        w = rne(a) | (rne(b) >> 16)
        return jnp.transpose(lax.bitcast_convert_type(w, jnp.float32),
                             (1, 0))

    out_ref[0] = jnp.concatenate(
        [pk(ult_ref[...], umt_ref[0]), pk(umt_ref[1], umt_ref[2])], axis=1)
    out_ref[1] = jnp.concatenate(
        [pk(ilt_ref[...], sst_ref[0]), pk(sst_ref[1], sst_ref[2])], axis=1)


def _transpose_all(ult, ilt, umt, sst):
    nb = (NUM_USERS + TN - 1) // TN
    return pl.pallas_call(
        _transpose_body,
        grid=(nb,),
        in_specs=[
            pl.BlockSpec((EMB, TN), lambda i: (0, i)),
            pl.BlockSpec((EMB, TN), lambda i: (0, i)),
            pl.BlockSpec((T, EMB, TN), lambda i: (0, 0, i)),
            pl.BlockSpec((T, EMB, TN), lambda i: (0, 0, i)),
        ],
        out_specs=pl.BlockSpec((2, TN, PW), lambda i: (0, i, 0)),
        out_shape=jax.ShapeDtypeStruct((2, NUM_USERS, PW), jnp.float32),
    )(ult, ilt, umt, sst)


BLK = 4096


def _tc_body(rows_ref, m_ref, out_ref):
    def unpk(r, k):
        w = lax.bitcast_convert_type(
            r[:, k * EMB:(k + 1) * EMB], jnp.uint32)
        hi = lax.bitcast_convert_type(w & jnp.uint32(0xFFFF0000),
                                      jnp.float32)
        lo = lax.bitcast_convert_type(w << 16, jnp.float32)
        return hi, lo

    ru = rows_ref[0]
    ri = rows_ref[1]
    u, p0 = unpk(ru, 0)
    p1, p2 = unpk(ru, 1)
    i, s0 = unpk(ri, 0)
    s1, s2 = unpk(ri, 1)
    p = (p0, p1, p2)
    s = (s0, s1, s2)
    acc = LAMB * jnp.sum(u * i, axis=-1, keepdims=True)
    w = (1.0 - LAMB) / T
    for t in range(T):
        proj = lax.dot_general(
            p[t].astype(jnp.bfloat16),
            m_ref[t].astype(jnp.bfloat16),
            (((1,), (0,)), ((), ())),
            preferred_element_type=jnp.float32,
        )
        acc = acc + w * jnp.sum(proj * s[t], axis=-1, keepdims=True)
    out_ref[...] = acc


def kernel(user_idx, item_idx, user_latent, item_latent, s_item_list,
           user_mean_emb, M_t):
    ui = user_idx.astype(jnp.int32)
    ii = item_idx.astype(jnp.int32)
    ult = jnp.swapaxes(user_latent, 0, 1)
    ilt = jnp.swapaxes(item_latent, 0, 1)
    umt = jnp.transpose(user_mean_emb, (0, 2, 1))
    sst = jnp.transpose(s_item_list, (0, 2, 1))
    planes = _transpose_all(ult, ilt, umt, sst)
    planes_flat = planes.reshape(2 * NUM_USERS, PW)
    idx_all = jnp.concatenate(
        [ui[None], ii[None] + jnp.int32(NUM_USERS)], axis=0
    ).reshape(2, NW, NCHUNK, CHUNK)

    rows = _sc_gather(planes_flat, idx_all)

    score2 = pl.pallas_call(
        _tc_body,
        grid=(BATCH // BLK,),
        in_specs=[
            pl.BlockSpec((2, BLK, PW), lambda i: (0, i, 0)),
            pl.BlockSpec((T, EMB, EMB), lambda i: (0, 0, 0)),
        ],
        out_specs=pl.BlockSpec((BLK, 1), lambda i: (i, 0)),
        out_shape=jax.ShapeDtypeStruct((BATCH, 1), jnp.float32),
    )(rows, M_t)
    return score2[:, 0]
